# baseline (speedup 1.0000x reference)
import jax
import jax.numpy as jnp
from jax import lax
from jax.experimental import pallas as pl
from jax.experimental.pallas import tpu as pltpu
from jax.experimental.pallas import tpu_sc as plsc

_B = 16384
_D = 128
_L = 16
_NS = 16
_BH = _B // 2
_BPW = _BH // _NS
_C = 128
_NCHUNK = _BPW // _C


def _half_body(h_hbm, t_hbm, r_hbm, tab_hbm, out_hbm,
               idx_v, h_v, t_v, rel_v, out_v,
               isem, sem0, sem1, osem):
    wid = lax.axis_index("s")
    base = wid * _BPW
    lane = lax.iota(jnp.int32, _L)
    last = lane == (_L - 1)
    sems = (sem0, sem1)

    idx_copies = [
        pltpu.async_copy(r_hbm.at[pl.ds(base + c * _C, _C)], idx_v.at[c], isem)
        for c in range(_NCHUNK)
    ]
    for cp in idx_copies:
        cp.wait()

    def fetch(c, b):
        off = base + c * _C
        return [
            pltpu.async_copy(tab_hbm.at[idx_v.at[c]], rel_v.at[b], sems[b]),
            pltpu.async_copy(h_hbm.at[pl.ds(off, _C), :], h_v.at[b], sems[b]),
            pltpu.async_copy(t_hbm.at[pl.ds(off, _C), :], t_v.at[b], sems[b]),
        ]

    pending = fetch(0, 0)
    out_copies = [None, None]
    for c in range(_NCHUNK):
        b = c % 2
        nxt = fetch(c + 1, 1 - b) if c + 1 < _NCHUNK else None
        for cp in pending:
            cp.wait()
        pending = nxt
        if out_copies[b] is not None:
            out_copies[b].wait()

        def row_body(i, carry, _b=b):
            accs = [jnp.zeros((_L,), jnp.float32) for _ in range(4)]
            for j in range(_D // _L):
                sl = pl.ds(j * _L, _L)
                accs[j % 4] = accs[j % 4] + (
                    h_v[_b, i, sl] * rel_v[_b, i, sl]) * t_v[_b, i, sl]
            acc = (accs[0] + accs[1]) + (accs[2] + accs[3])
            cum = plsc.cumsum(acc)
            plsc.store_compressed(out_v.at[_b, pl.ds(i, _L)], cum, mask=last)
            return carry

        lax.fori_loop(0, _C, row_body, 0, unroll=4)
        out_copies[b] = pltpu.async_copy(
            out_v.at[b, pl.ds(0, _C)], out_hbm.at[pl.ds(base + c * _C, _C)],
            osem)
    for cp in out_copies:
        if cp is not None:
            cp.wait()


def _make_half():
    mesh = plsc.VectorSubcoreMesh(
        core_axis_name="c", subcore_axis_name="s", num_cores=1)
    return pl.kernel(
        _half_body,
        out_type=jax.ShapeDtypeStruct((_BH,), jnp.float32),
        mesh=mesh,
        compiler_params=pltpu.CompilerParams(needs_layout_passes=False),
        scratch_types=[
            pltpu.VMEM((_NCHUNK, _C), jnp.int32),
            pltpu.VMEM((2, _C, _D), jnp.float32),
            pltpu.VMEM((2, _C, _D), jnp.float32),
            pltpu.VMEM((2, _C, _D), jnp.float32),
            pltpu.VMEM((2, _C + _L), jnp.float32),
            pltpu.SemaphoreType.DMA,
            pltpu.SemaphoreType.DMA,
            pltpu.SemaphoreType.DMA,
            pltpu.SemaphoreType.DMA,
        ],
    )


@jax.jit
def _dist_mult(h, t, r, rel_emb):
    run = _make_half()
    lo = run(h[:_BH], t[:_BH], r[:_BH], rel_emb)
    hi = run(h[_BH:], t[_BH:], r[_BH:], rel_emb)
    return jnp.concatenate([lo, hi])


def kernel(h, t, r, rel_emb):
    return _dist_mult(h, t, r.astype(jnp.int32), rel_emb)


# speedup vs baseline: 1.9976x; 1.9976x over previous
import jax
import jax.numpy as jnp
from jax import lax
from jax.experimental import pallas as pl
from jax.experimental.pallas import tpu as pltpu
from jax.experimental.pallas import tpu_sc as plsc

_B = 16384
_D = 128
_L = 16
_NW = 32
_BPW = _B // _NW
_C = 128
_NCHUNK = _BPW // _C


def _dist_mult_body(h_hbm, t_hbm, r_hbm, tab_hbm, out_hbm,
                    idx_v, h_v, t_v, rel_v, out_v,
                    isem, sem0, sem1, osem):
    cid = lax.axis_index("c")
    sid = lax.axis_index("s")
    wid = sid * 2 + cid
    base = wid * _BPW
    lane = lax.iota(jnp.int32, _L)
    last = lane == (_L - 1)
    sems = (sem0, sem1)

    idx_copies = [
        pltpu.async_copy(r_hbm.at[pl.ds(base + c * _C, _C)], idx_v.at[c], isem)
        for c in range(_NCHUNK)
    ]
    for cp in idx_copies:
        cp.wait()

    def fetch(c, b):
        off = base + c * _C
        return [
            pltpu.async_copy(tab_hbm.at[idx_v.at[c]], rel_v.at[b], sems[b]),
            pltpu.async_copy(h_hbm.at[pl.ds(off, _C), :], h_v.at[b], sems[b]),
            pltpu.async_copy(t_hbm.at[pl.ds(off, _C), :], t_v.at[b], sems[b]),
        ]

    pending = fetch(0, 0)
    out_copies = [None, None]
    for c in range(_NCHUNK):
        b = c % 2
        nxt = fetch(c + 1, 1 - b) if c + 1 < _NCHUNK else None
        for cp in pending:
            cp.wait()
        pending = nxt
        if out_copies[b] is not None:
            out_copies[b].wait()

        @plsc.parallel_loop(0, _C, unroll=4)
        def row_body(i, _b=b):
            accs = [jnp.zeros((_L,), jnp.float32) for _ in range(4)]
            for j in range(_D // _L):
                sl = pl.ds(j * _L, _L)
                accs[j % 4] = accs[j % 4] + (
                    h_v[_b, i, sl] * rel_v[_b, i, sl]) * t_v[_b, i, sl]
            acc = (accs[0] + accs[1]) + (accs[2] + accs[3])
            cum = plsc.cumsum(acc)
            plsc.store_compressed(out_v.at[_b, pl.ds(i, _L)], cum, mask=last)
        out_copies[b] = pltpu.async_copy(
            out_v.at[b, pl.ds(0, _C)], out_hbm.at[pl.ds(base + c * _C, _C)],
            osem)
    for cp in out_copies:
        if cp is not None:
            cp.wait()


@jax.jit
def _dist_mult(h, t, r, rel_emb):
    mesh = plsc.VectorSubcoreMesh(core_axis_name="c", subcore_axis_name="s")
    run = pl.kernel(
        _dist_mult_body,
        out_type=jax.ShapeDtypeStruct((_B,), jnp.float32),
        mesh=mesh,
        compiler_params=pltpu.CompilerParams(needs_layout_passes=False),
        scratch_types=[
            pltpu.VMEM((_NCHUNK, _C), jnp.int32),
            pltpu.VMEM((2, _C, _D), jnp.float32),
            pltpu.VMEM((2, _C, _D), jnp.float32),
            pltpu.VMEM((2, _C, _D), jnp.float32),
            pltpu.VMEM((2, _C + _L), jnp.float32),
            pltpu.SemaphoreType.DMA,
            pltpu.SemaphoreType.DMA,
            pltpu.SemaphoreType.DMA,
            pltpu.SemaphoreType.DMA,
        ],
    )
    return run(h, t, r, rel_emb)


def kernel(h, t, r, rel_emb):
    return _dist_mult(h, t, r.astype(jnp.int32), rel_emb)
